# Initial kernel scaffold; baseline (speedup 1.0000x reference)
import functools

import jax
import jax.numpy as jnp
from jax import lax
from jax.experimental import pallas as pl
from jax.experimental.pallas import tpu as pltpu
from jax.experimental.pallas import tpu_sc as plsc

_VOCAB = 1000000
_D = 32
_BATCH = 16384
_FIELDS = 100
_B = _BATCH * _FIELDS
_NW = 32
_PER_W = _B // _NW
_CHUNK = 1024
_NCHUNK = _PER_W // _CHUNK


def _emb_body(idx_hbm, table_hbm, out_hbm, idx_v, rows_v, sem):
    wid = lax.axis_index("s") * 2 + lax.axis_index("c")
    base = wid * _PER_W

    def body(g, carry):
        off = base + g * _CHUNK
        pltpu.sync_copy(idx_hbm.at[pl.ds(off, _CHUNK)], idx_v)
        pltpu.async_copy(table_hbm.at[idx_v], rows_v, sem).wait()
        pltpu.sync_copy(rows_v, out_hbm.at[pl.ds(off, _CHUNK)])
        return carry

    lax.fori_loop(0, _NCHUNK, body, 0)


def kernel(input, weight):
    idx = input.reshape(-1).astype(jnp.int32)
    mesh = plsc.VectorSubcoreMesh(core_axis_name="c", subcore_axis_name="s")
    out = pl.kernel(
        _emb_body,
        out_type=jax.ShapeDtypeStruct((_B, _D), jnp.float32),
        mesh=mesh,
        scratch_types=[
            pltpu.VMEM((_CHUNK,), jnp.int32),
            pltpu.VMEM((_CHUNK, _D), jnp.float32),
            pltpu.SemaphoreType.DMA,
        ],
    )(idx, weight)
    return out.reshape(_BATCH, _FIELDS, _D)


# speedup vs baseline: 1.1018x; 1.1018x over previous
import functools

import jax
import jax.numpy as jnp
from jax import lax
from jax.experimental import pallas as pl
from jax.experimental.pallas import tpu as pltpu
from jax.experimental.pallas import tpu_sc as plsc

_VOCAB = 1000000
_D = 32
_BATCH = 16384
_FIELDS = 100
_B = _BATCH * _FIELDS
_NW = 32
_PER_W = _B // _NW
_CHUNK = 1024
_NCHUNK = _PER_W // _CHUNK


def _emb_body(idx_hbm, table_hbm, out_hbm, idx_v, rows_v, sem):
    wid = lax.axis_index("s") * 2 + lax.axis_index("c")
    base = wid * _PER_W

    def body(g, carry):
        off = base + g * _CHUNK
        pltpu.sync_copy(idx_hbm.at[pl.ds(off, _CHUNK)], idx_v)
        pltpu.async_copy(table_hbm.at[idx_v], rows_v, sem).wait()
        pltpu.sync_copy(rows_v, out_hbm.at[pl.ds(off, _CHUNK)])
        return carry

    lax.fori_loop(0, _NCHUNK, body, 0)


def kernel(input, weight):
    idx = input.reshape(-1).astype(jnp.int32)
    mesh = plsc.VectorSubcoreMesh(core_axis_name="c", subcore_axis_name="s")
    out = pl.kernel(
        _emb_body,
        out_type=jax.ShapeDtypeStruct((_B, _D), jnp.float32),
        mesh=mesh,
        scratch_types=[
            pltpu.VMEM((_CHUNK,), jnp.int32),
            pltpu.VMEM((_CHUNK, _D), jnp.float32),
            pltpu.SemaphoreType.DMA,
        ],
        compiler_params=pltpu.CompilerParams(use_tc_tiling_on_sc=False),
    )(idx, weight)
    return out.reshape(_BATCH, _FIELDS, _D)


# speedup vs baseline: 1.1124x; 1.0096x over previous
import functools

import jax
import jax.numpy as jnp
from jax import lax
from jax.experimental import pallas as pl
from jax.experimental.pallas import tpu as pltpu
from jax.experimental.pallas import tpu_sc as plsc

_VOCAB = 1000000
_D = 32
_BATCH = 16384
_FIELDS = 100
_B = _BATCH * _FIELDS
_NW = 32
_PER_W = _B // _NW
_CHUNK = 1024
_NCHUNK = _PER_W // _CHUNK
_NBUF = 2
_NWAVE = _NCHUNK // _NBUF


def _emb_body(idx_hbm, table_hbm, out_hbm, idx_v, rows0, rows1,
              sg0, sg1, sw0, sw1):
    wid = lax.axis_index("s") * 2 + lax.axis_index("c")
    base = wid * _PER_W
    rows = (rows0, rows1)
    sg = (sg0, sg1)
    sw = (sw0, sw1)

    pltpu.sync_copy(idx_hbm.at[pl.ds(base, _PER_W)], idx_v)

    def gather(g, b):
        pltpu.async_copy(
            table_hbm.at[idx_v.at[pl.ds(g * _CHUNK, _CHUNK)]], rows[b], sg[b])

    def wait_gather(b):
        pltpu.make_async_copy(
            table_hbm.at[idx_v.at[pl.ds(0, _CHUNK)]], rows[b], sg[b]).wait()

    def writeback(g, b):
        pltpu.async_copy(
            rows[b], out_hbm.at[pl.ds(base + g * _CHUNK, _CHUNK)], sw[b])

    def wait_writeback(b):
        pltpu.make_async_copy(
            rows[b], out_hbm.at[pl.ds(base, _CHUNK)], sw[b]).wait()

    for b in range(_NBUF):
        gather(b, b)

    def body(w, carry):
        for b in range(_NBUF):
            g = w * _NBUF + b
            wait_gather(b)
            writeback(g, b)
            wait_writeback(b)
            gather(g + _NBUF, b)
        return carry

    lax.fori_loop(0, _NWAVE - 1, body, 0)

    for b in range(_NBUF):
        g = (_NWAVE - 1) * _NBUF + b
        wait_gather(b)
        writeback(g, b)
        wait_writeback(b)


def kernel(input, weight):
    idx = input.reshape(-1).astype(jnp.int32)
    mesh = plsc.VectorSubcoreMesh(core_axis_name="c", subcore_axis_name="s")
    out = pl.kernel(
        _emb_body,
        out_type=jax.ShapeDtypeStruct((_B, _D), jnp.float32),
        mesh=mesh,
        scratch_types=[
            pltpu.VMEM((_PER_W,), jnp.int32),
            pltpu.VMEM((_CHUNK, _D), jnp.float32),
            pltpu.VMEM((_CHUNK, _D), jnp.float32),
            pltpu.SemaphoreType.DMA,
            pltpu.SemaphoreType.DMA,
            pltpu.SemaphoreType.DMA,
            pltpu.SemaphoreType.DMA,
        ],
        compiler_params=pltpu.CompilerParams(use_tc_tiling_on_sc=False),
    )(idx, weight)
    return out.reshape(_BATCH, _FIELDS, _D)


# speedup vs baseline: 4.0242x; 3.6174x over previous
import functools

import jax
import jax.numpy as jnp
from jax import lax
from jax.experimental import pallas as pl
from jax.experimental.pallas import tpu as pltpu
from jax.experimental.pallas import tpu_sc as plsc

_VOCAB = 1000000
_D = 32
_BATCH = 16384
_FIELDS = 100
_NW = 32
_BW = _BATCH // _NW


def _emb_body(idx_hbm, table_hbm, out_hbm, idx_v, rows0, rows1, t0, t1,
              sg0, sg1, sw0, sw1):
    wid = lax.axis_index("s") * 2 + lax.axis_index("c")
    b0 = wid * _BW
    rows = (rows0, rows1)
    tbuf = (t0, t1)
    sg = (sg0, sg1)
    sw = (sw0, sw1)

    iota_lo = lax.iota(jnp.int32, 16)
    iota_hi = iota_lo + 16

    pltpu.sync_copy(idx_hbm.at[:, pl.ds(b0, _BW)], idx_v)

    def gather(f, b):
        pltpu.async_copy(table_hbm.at[idx_v.at[f]], rows[b], sg[b])

    def wait_gather(b):
        pltpu.make_async_copy(table_hbm.at[idx_v.at[0]], rows[b], sg[b]).wait()

    def writeback(f, b):
        pltpu.async_copy(tbuf[b], out_hbm.at[f, :, pl.ds(b0, _BW)], sw[b])

    def wait_writeback(b):
        pltpu.make_async_copy(
            tbuf[b], out_hbm.at[0, :, pl.ds(b0, _BW)], sw[b]).wait()

    def transpose(b):
        rv = rows[b]
        tv = tbuf[b]

        def tbody(j, carry):
            v0 = rv[j, pl.ds(0, 16)]
            v1 = rv[j, pl.ds(16, 16)]
            jj = jnp.full((16,), 0, jnp.int32) + j
            plsc.store_scatter(tv, [iota_lo, jj], v0)
            plsc.store_scatter(tv, [iota_hi, jj], v1)
            return carry

        lax.fori_loop(0, _BW, tbody, 0, unroll=4)

    for b in range(2):
        gather(b, b)
    for b in range(2):
        wait_gather(b)
        transpose(b)
        gather(b + 2, b)
        writeback(b, b)

    def wave(w, carry):
        for b in range(2):
            f = 2 * w + b
            wait_gather(b)
            wait_writeback(b)
            transpose(b)
            gather(f + 2, b)
            writeback(f, b)
        return carry

    lax.fori_loop(1, _FIELDS // 2 - 1, wave, 0)

    for b in range(2):
        f = _FIELDS - 2 + b
        wait_gather(b)
        wait_writeback(b)
        transpose(b)
        writeback(f, b)
    for b in range(2):
        wait_writeback(b)


def kernel(input, weight):
    idx_fb = input.T
    mesh = plsc.VectorSubcoreMesh(core_axis_name="c", subcore_axis_name="s")
    out = pl.kernel(
        _emb_body,
        out_type=jax.ShapeDtypeStruct((_FIELDS, _D, _BATCH), jnp.float32),
        mesh=mesh,
        scratch_types=[
            pltpu.VMEM((_FIELDS, _BW), jnp.int32),
            pltpu.VMEM((_BW, _D), jnp.float32),
            pltpu.VMEM((_BW, _D), jnp.float32),
            pltpu.VMEM((_D, _BW), jnp.float32),
            pltpu.VMEM((_D, _BW), jnp.float32),
            pltpu.SemaphoreType.DMA,
            pltpu.SemaphoreType.DMA,
            pltpu.SemaphoreType.DMA,
            pltpu.SemaphoreType.DMA,
        ],
        compiler_params=pltpu.CompilerParams(
            use_tc_tiling_on_sc=False, needs_layout_passes=False),
    )(idx_fb, weight)
    return out.transpose(2, 0, 1)
